# baseline (speedup 1.0000x reference)
import functools
import math

import jax
import jax.numpy as jnp
from jax import lax
from jax.experimental import pallas as pl
from jax.experimental.pallas import tpu as pltpu, tpu_sc as plsc

_VOC_SIZE = 1000000
_DIM = 64
_B = 4096
_L = 200
_COE = math.sqrt(_DIM)

_NW = 32
_N_TOTAL = _B * _L
_PER_W = _N_TOTAL // _NW
_CHUNK = 512
_N_CHUNKS = _PER_W // _CHUNK


def _emb_body(table_hbm, x_hbm, out_hbm, idx_v, rows_v, sem):
    nc = 2
    wid = lax.axis_index("s") * nc + lax.axis_index("c")
    wbase = wid * _PER_W

    def chunk_body(i, carry):
        base = wbase + i * _CHUNK
        pltpu.sync_copy(x_hbm.at[pl.ds(base, _CHUNK)], idx_v)
        pltpu.async_copy(table_hbm.at[idx_v], rows_v, sem).wait()

        def row_body(r, c):
            for j in range(_DIM // 16):
                s = pl.ds(j * 16, 16)
                rows_v[r, s] = rows_v[r, s] * _COE
            return c

        lax.fori_loop(0, _CHUNK, row_body, 0)
        pltpu.sync_copy(rows_v, out_hbm.at[pl.ds(base, _CHUNK)])
        return carry

    lax.fori_loop(0, _N_CHUNKS, chunk_body, 0)


@jax.jit
def _emb(x_flat, table):
    mesh = plsc.VectorSubcoreMesh(core_axis_name="c", subcore_axis_name="s")
    f = functools.partial(
        pl.kernel,
        out_type=jax.ShapeDtypeStruct((_N_TOTAL, _DIM), jnp.float32),
        mesh=mesh,
        scratch_types=[
            pltpu.VMEM((_CHUNK,), jnp.int32),
            pltpu.VMEM((_CHUNK, _DIM), jnp.float32),
            pltpu.SemaphoreType.DMA,
        ],
        compiler_params=pltpu.CompilerParams(use_tc_tiling_on_sc=False),
    )(_emb_body)
    return f(table, x_flat)


def kernel(x, table):
    x_flat = x.reshape(-1).astype(jnp.int32)
    out = _emb(x_flat, table)
    return out.reshape(_B, _L, _DIM)


# speedup vs baseline: 1.1363x; 1.1363x over previous
import functools
import math

import jax
import jax.numpy as jnp
from jax import lax
from jax.experimental import pallas as pl
from jax.experimental.pallas import tpu as pltpu, tpu_sc as plsc

_VOC_SIZE = 1000000
_DIM = 64
_B = 4096
_L = 200
_COE = math.sqrt(_DIM)

_NW = 32
_N_TOTAL = _B * _L
_PER_W = _N_TOTAL // _NW
_CHUNK = 320
_N_CHUNKS = _PER_W // _CHUNK
_NBUF = 4


def _emb_body(table_hbm, x_hbm, out_hbm, idx_v, rows_v,
              g0, g1, g2, g3, s0, s1, s2, s3):
    gs = [g0, g1, g2, g3]
    ss = [s0, s1, s2, s3]
    wid = lax.axis_index("s") * 2 + lax.axis_index("c")
    wbase = wid * _PER_W
    pltpu.sync_copy(x_hbm.at[pl.ds(wbase, _PER_W)], idx_v)

    def g_desc(c, b):
        return pltpu.make_async_copy(
            table_hbm.at[idx_v.at[pl.ds(c * _CHUNK, _CHUNK)]],
            rows_v.at[b], gs[b])

    def s_desc(c, b):
        return pltpu.make_async_copy(
            rows_v.at[b],
            out_hbm.at[pl.ds(wbase + c * _CHUNK, _CHUNK)], ss[b])

    def scale(b):
        @plsc.parallel_loop(0, _CHUNK, 1, unroll=8)
        def _(r):
            for j in range(_DIM // 16):
                sl = pl.ds(j * 16, 16)
                rows_v[b, r, sl] = rows_v[b, r, sl] * _COE

    def step(c, b, prefetch, with_store_wait):
        g_desc(c, b).wait()
        scale(b)
        s_desc(c, b).start()
        if prefetch:
            b2 = (b + 2) % _NBUF
            if with_store_wait:
                s_desc(c, b2).wait()
            g_desc(c + 2, b2).start()

    g_desc(0, 0).start()
    g_desc(1, 1).start()
    step(0, 0, True, False)
    step(1, 1, True, False)

    def group(g, carry):
        c0 = 2 + g * _NBUF
        for k in range(_NBUF):
            step(c0 + k, (2 + k) % _NBUF, True, True)
        return carry

    lax.fori_loop(0, (_N_CHUNKS - 4) // _NBUF, group, 0)

    step(_N_CHUNKS - 2, (_N_CHUNKS - 2) % _NBUF, False, False)
    step(_N_CHUNKS - 1, (_N_CHUNKS - 1) % _NBUF, False, False)

    for b in range(_NBUF):
        s_desc(0, b).wait()


@jax.jit
def _emb(x_flat, table):
    mesh = plsc.VectorSubcoreMesh(core_axis_name="c", subcore_axis_name="s")
    f = functools.partial(
        pl.kernel,
        out_type=jax.ShapeDtypeStruct((_N_TOTAL, _DIM), jnp.float32),
        mesh=mesh,
        scratch_types=[
            pltpu.VMEM((_PER_W,), jnp.int32),
            pltpu.VMEM((_NBUF, _CHUNK, _DIM), jnp.float32),
        ] + [pltpu.SemaphoreType.DMA] * (2 * _NBUF),
        compiler_params=pltpu.CompilerParams(use_tc_tiling_on_sc=False),
    )(_emb_body)
    return f(table, x_flat)


def kernel(x, table):
    x_flat = x.reshape(-1).astype(jnp.int32)
    out = _emb(x_flat, table)
    return out.reshape(_B, _L, _DIM)
